# baseline (speedup 1.0000x reference)
import functools

import jax
import jax.numpy as jnp
from jax import lax
from jax.experimental import pallas as pl
from jax.experimental.pallas import tpu as pltpu
from jax.experimental.pallas import tpu_sc as plsc

_MAX_VAL = 52000.0
_MIN_VAL = -53000.0
_LEVELS = 1000
_B, _T, _C, _D = 64, 50, 16, 1024
_NCORE, _NSUB = 2, 16
_NW = _NCORE * _NSUB
_NWIN = _T - 3
_W = _D // 32
_ROWW = _W + 1
_NROWS = _LEVELS + _C + _T


def _i16(v):
    return jnp.full((16,), v, jnp.int32)


def _pack_body(x_ref, wlo_ref, whi_ref, o_ref):
    x = x_ref[...].astype(jnp.bfloat16)
    lo = lax.dot(x, wlo_ref[...], preferred_element_type=jnp.float32)
    hi = lax.dot(x, whi_ref[...], preferred_element_type=jnp.float32)
    plo = ((65535.0 - lo) * 0.5).astype(jnp.int32)
    phi = ((65535.0 - hi) * 0.5).astype(jnp.int32)
    o_ref[...] = jnp.bitwise_or(plo, jnp.left_shift(phi, 16))


def _add_nums(a, b):
    out, carry = [], None
    for i in range(max(len(a), len(b))):
        terms = [t for t in (a[i] if i < len(a) else None,
                             b[i] if i < len(b) else None, carry)
                 if t is not None]
        if len(terms) == 3:
            x, y, z = terms
            s = x ^ y
            carry, s = (x & y) | (z & s), s ^ z
        elif len(terms) == 2:
            x, y = terms
            s, carry = x ^ y, x & y
        else:
            s, carry = terms[0], None
        out.append(s)
    if carry is not None:
        out.append(carry)
    return out


def _popcount_planes(xs):
    nums = [[x] for x in xs]
    while len(nums) > 1:
        nxt = [_add_nums(nums[i], nums[i + 1])
               for i in range(0, len(nums) - 1, 2)]
        if len(nums) % 2:
            nxt.append(nums[-1])
        nums = nxt
    return nums[0]


def _tec_body(idx_hbm, pkb_hbm, out_hbm,
              inp_v, lw_v, ch_v, ft_v, samp_v, r1_v, r2_v, r3_v, out_v, sem):
    wid = lax.axis_index("s") * _NCORE + lax.axis_index("c")
    pltpu.sync_copy(pkb_hbm.at[pl.ds(0, _LEVELS * _W)], lw_v)
    pltpu.sync_copy(pkb_hbm.at[pl.ds(_LEVELS * _W, _C * _W)], ch_v)
    pltpu.sync_copy(pkb_hbm.at[pl.ds((_LEVELS + _C) * _W, _T * _W)], ft_v)
    iota = lax.iota(jnp.int32, 16)

    def do_batch(b):
        pltpu.sync_copy(idx_hbm.at[pl.ds(b * (_T * _C), _T * _C)], inp_v)

        def do_t(t, carry):
            idx = inp_v[pl.ds(t * _C, 16)]
            addr = idx * _W
            hi_words = None
            for h in range(2):
                xs = []
                for c in range(_C):
                    ac = addr.at[_i16(c)].get(mode="promise_in_bounds")
                    lww = plsc.load_gather(lw_v, [ac + (iota + 16 * h)])
                    xs.append(lww ^ ch_v[pl.ds(c * _W + 16 * h, 16)])
                pa = _popcount_planes(xs[:8])
                pb = _popcount_planes(xs[8:])
                c1 = pa[0] & pb[0]
                c2 = (pa[1] & pb[1]) | (c1 & (pa[1] ^ pb[1]))
                c3 = (pa[2] & pb[2]) | (c2 & (pa[2] ^ pb[2]))
                neg = pa[3] | pb[3] | c3
                sw = neg ^ ft_v[pl.ds(t * _W + 16 * h, 16)]
                samp_v[pl.ds(t * _ROWW + 1 + 16 * h, 16)] = sw
                if h == 1:
                    hi_words = sw
            w31 = hi_words.at[_i16(15)].get(mode="promise_in_bounds")
            plsc.store_scatter(samp_v, [jnp.zeros((16,), jnp.int32) + t * _ROWW],
                               w31, mask=iota < 1)
            for h in range(2):
                a = samp_v[pl.ds(t * _ROWW + 1 + 16 * h, 16)]
                bb = samp_v[pl.ds(t * _ROWW + 16 * h, 16)]
                for s, rv in ((1, r1_v), (2, r2_v), (3, r3_v)):
                    r = (lax.shift_left(a, _i16(s))
                         | lax.shift_right_logical(bb, _i16(32 - s)))
                    rv[pl.ds(t * _W + 16 * h, 16)] = r
            return carry

        lax.fori_loop(0, _T, do_t, 0)

        for h in range(2):
            def win(t0, planes):
                g = (r3_v[pl.ds(t0 * _W + 16 * h, 16)]
                     ^ r2_v[pl.ds((t0 + 1) * _W + 16 * h, 16)]
                     ^ r1_v[pl.ds((t0 + 2) * _W + 16 * h, 16)]
                     ^ samp_v[pl.ds((t0 + 3) * _ROWW + 1 + 16 * h, 16)])
                out_p = []
                c = g
                for i in range(6):
                    out_p.append(planes[i] ^ c)
                    c = planes[i] & c
                return tuple(out_p)

            z = jnp.zeros((16,), jnp.int32)
            planes = lax.fori_loop(0, _NWIN, win, (z, z, z, z, z, z))
            negw = planes[5] | (planes[4] & planes[3])
            for wslot in range(16):
                w = negw.at[_i16(wslot)].get(mode="promise_in_bounds")
                wi = 16 * h + wslot
                b0 = lax.shift_right_logical(w, iota) & 1
                out_v[pl.ds(32 * wi, 16)] = jnp.where(b0 == 1, -1.0, 1.0)
                b1 = lax.shift_right_logical(w, iota + 16) & 1
                out_v[pl.ds(32 * wi + 16, 16)] = jnp.where(b1 == 1, -1.0, 1.0)

        pltpu.sync_copy(out_v, out_hbm.at[b])

    do_batch(wid)
    do_batch(wid + _NW)


@jax.jit
def _encode(inp, lw, ch, ft):
    x = jnp.round((inp - _MIN_VAL) / (_MAX_VAL - _MIN_VAL) * (_LEVELS - 1))
    idx = jnp.clip(x, 0, _LEVELS - 1).astype(jnp.int32)

    d = jnp.arange(_D)
    j, k = d // 32, d % 32
    onehot = (j[:, None] == jnp.arange(_W)[None, :]).astype(jnp.float32)
    wlo = (onehot * jnp.where(k < 16, jnp.left_shift(1, jnp.minimum(k, 15)),
                              0).astype(jnp.float32)[:, None]
           ).astype(jnp.bfloat16)
    whi = (onehot * jnp.where(k >= 16, jnp.left_shift(1, k - 16),
                              0).astype(jnp.float32)[:, None]
           ).astype(jnp.bfloat16)
    codes = jnp.concatenate([lw, ch, ft], axis=0)
    packed = pl.pallas_call(
        _pack_body,
        grid=(17,),
        in_specs=[
            pl.BlockSpec((64, _D), lambda i: (i, 0)),
            pl.BlockSpec((_D, _W), lambda i: (0, 0)),
            pl.BlockSpec((_D, _W), lambda i: (0, 0)),
        ],
        out_specs=pl.BlockSpec((64, _W), lambda i: (i, 0)),
        out_shape=jax.ShapeDtypeStruct((_NROWS, _W), jnp.int32),
        compiler_params=pltpu.CompilerParams(
            dimension_semantics=("arbitrary",)),
    )(codes, wlo, whi)
    pkb = packed.reshape(-1)

    mesh = plsc.VectorSubcoreMesh(core_axis_name="c", subcore_axis_name="s")
    f = functools.partial(
        pl.kernel,
        mesh=mesh,
        compiler_params=pltpu.CompilerParams(needs_layout_passes=False),
        out_type=jax.ShapeDtypeStruct((_B, _D), jnp.float32),
        scratch_types=[
            pltpu.VMEM((_T * _C,), jnp.int32),
            pltpu.VMEM((_LEVELS * _W,), jnp.int32),
            pltpu.VMEM((_C * _W,), jnp.int32),
            pltpu.VMEM((_T * _W,), jnp.int32),
            pltpu.VMEM((_T * _ROWW,), jnp.int32),
            pltpu.VMEM((_T * _W,), jnp.int32),
            pltpu.VMEM((_T * _W,), jnp.int32),
            pltpu.VMEM((_T * _W,), jnp.int32),
            pltpu.VMEM((_D,), jnp.float32),
            pltpu.SemaphoreType.DMA,
        ],
    )(_tec_body)
    return f(idx.reshape(-1), pkb)


def kernel(input, level_weight, channel_weight, feature_weight):
    return _encode(input, level_weight, channel_weight, feature_weight)


# speedup vs baseline: 1.0793x; 1.0793x over previous
import functools

import jax
import jax.numpy as jnp
from jax import lax
from jax.experimental import pallas as pl
from jax.experimental.pallas import tpu as pltpu
from jax.experimental.pallas import tpu_sc as plsc

_MAX_VAL = 52000.0
_MIN_VAL = -53000.0
_LEVELS = 1000
_B, _T, _C, _D = 64, 50, 16, 1024
_NCORE, _NSUB = 2, 16
_NW = _NCORE * _NSUB
_NWIN = _T - 3
_W = _D // 32
_ROWW = _W + 1
_NROWS = _LEVELS + _C + _T


def _i16(v):
    return jnp.full((16,), v, jnp.int32)


def _pack_body(x_ref, wlo_ref, whi_ref, o_ref):
    x = x_ref[...].astype(jnp.bfloat16)
    lo = lax.dot(x, wlo_ref[...], preferred_element_type=jnp.float32)
    hi = lax.dot(x, whi_ref[...], preferred_element_type=jnp.float32)
    plo = ((65535.0 - lo) * 0.5).astype(jnp.int32)
    phi = ((65535.0 - hi) * 0.5).astype(jnp.int32)
    o_ref[...] = jnp.bitwise_or(plo, jnp.left_shift(phi, 16))


def _add_nums(a, b):
    out, carry = [], None
    for i in range(max(len(a), len(b))):
        terms = [t for t in (a[i] if i < len(a) else None,
                             b[i] if i < len(b) else None, carry)
                 if t is not None]
        if len(terms) == 3:
            x, y, z = terms
            s = x ^ y
            carry, s = (x & y) | (z & s), s ^ z
        elif len(terms) == 2:
            x, y = terms
            s, carry = x ^ y, x & y
        else:
            s, carry = terms[0], None
        out.append(s)
    if carry is not None:
        out.append(carry)
    return out


def _popcount_planes(xs):
    nums = [[x] for x in xs]
    while len(nums) > 1:
        nxt = [_add_nums(nums[i], nums[i + 1])
               for i in range(0, len(nums) - 1, 2)]
        if len(nums) % 2:
            nxt.append(nums[-1])
        nums = nxt
    return nums[0]


def _tec_body(idx_hbm, pkb_hbm, out_hbm,
              inp_v, lw_v, ch_v, ft_v, samp_v, r1_v, r2_v, r3_v, out_v, sem):
    wid = lax.axis_index("s") * _NCORE + lax.axis_index("c")
    pltpu.sync_copy(pkb_hbm.at[pl.ds(0, _LEVELS * _W)], lw_v)
    pltpu.sync_copy(pkb_hbm.at[pl.ds(_LEVELS * _W, _C * _W)], ch_v)
    pltpu.sync_copy(pkb_hbm.at[pl.ds((_LEVELS + _C) * _W, _T * _W)], ft_v)
    iota = lax.iota(jnp.int32, 16)

    def do_batch(b):
        pltpu.sync_copy(idx_hbm.at[pl.ds(b * (_T * _C), _T * _C)], inp_v)

        def do_t(t, carry):
            idx = inp_v[pl.ds(t * _C, 16)]
            addr = idx * _W
            hi_words = None
            for h in range(2):
                xs = []
                for c in range(_C):
                    ac = addr.at[_i16(c)].get(mode="promise_in_bounds")
                    lww = plsc.load_gather(lw_v, [ac + (iota + 16 * h)])
                    xs.append(lww ^ ch_v[pl.ds(c * _W + 16 * h, 16)])
                pa = _popcount_planes(xs[:8])
                pb = _popcount_planes(xs[8:])
                c1 = pa[0] & pb[0]
                c2 = (pa[1] & pb[1]) | (c1 & (pa[1] ^ pb[1]))
                c3 = (pa[2] & pb[2]) | (c2 & (pa[2] ^ pb[2]))
                neg = pa[3] | pb[3] | c3
                sw = neg ^ ft_v[pl.ds(t * _W + 16 * h, 16)]
                samp_v[pl.ds(t * _ROWW + 1 + 16 * h, 16)] = sw
                if h == 1:
                    hi_words = sw
            w31 = hi_words.at[_i16(15)].get(mode="promise_in_bounds")
            plsc.store_scatter(samp_v, [jnp.zeros((16,), jnp.int32) + t * _ROWW],
                               w31, mask=iota < 1)
            for h in range(2):
                a = samp_v[pl.ds(t * _ROWW + 1 + 16 * h, 16)]
                bb = samp_v[pl.ds(t * _ROWW + 16 * h, 16)]
                for s, rv in ((1, r1_v), (2, r2_v), (3, r3_v)):
                    r = (lax.shift_left(a, _i16(s))
                         | lax.shift_right_logical(bb, _i16(32 - s)))
                    rv[pl.ds(t * _W + 16 * h, 16)] = r
            return carry

        lax.fori_loop(0, _T, do_t, 0)

        for h in range(2):
            def win(t0, planes):
                g = (r3_v[pl.ds(t0 * _W + 16 * h, 16)]
                     ^ r2_v[pl.ds((t0 + 1) * _W + 16 * h, 16)]
                     ^ r1_v[pl.ds((t0 + 2) * _W + 16 * h, 16)]
                     ^ samp_v[pl.ds((t0 + 3) * _ROWW + 1 + 16 * h, 16)])
                out_p = []
                c = g
                for i in range(6):
                    out_p.append(planes[i] ^ c)
                    c = planes[i] & c
                return tuple(out_p)

            z = jnp.zeros((16,), jnp.int32)
            planes = lax.fori_loop(0, _NWIN, win, (z, z, z, z, z, z))
            negw = planes[5] | (planes[4] & planes[3])
            for wslot in range(16):
                w = negw.at[_i16(wslot)].get(mode="promise_in_bounds")
                wi = 16 * h + wslot
                b0 = lax.shift_right_logical(w, iota) & 1
                out_v[pl.ds(32 * wi, 16)] = jnp.where(b0 == 1, -1.0, 1.0)
                b1 = lax.shift_right_logical(w, iota + 16) & 1
                out_v[pl.ds(32 * wi + 16, 16)] = jnp.where(b1 == 1, -1.0, 1.0)

        pltpu.sync_copy(out_v, out_hbm.at[b])

    do_batch(wid)
    do_batch(wid + _NW)


@jax.jit
def _encode(inp, lw, ch, ft):
    x = jnp.round((inp - _MIN_VAL) / (_MAX_VAL - _MIN_VAL) * (_LEVELS - 1))
    idx = jnp.clip(x, 0, _LEVELS - 1).astype(jnp.int32)

    d = jnp.arange(_D)
    j, k = d // 32, d % 32
    onehot = (j[:, None] == jnp.arange(_W)[None, :]).astype(jnp.float32)
    wlo = (onehot * jnp.where(k < 16, jnp.left_shift(1, jnp.minimum(k, 15)),
                              0).astype(jnp.float32)[:, None]
           ).astype(jnp.bfloat16)
    whi = (onehot * jnp.where(k >= 16, jnp.left_shift(1, k - 16),
                              0).astype(jnp.float32)[:, None]
           ).astype(jnp.bfloat16)
    codes = jnp.concatenate([lw, ch, ft], axis=0)
    packed = pl.pallas_call(
        _pack_body,
        grid=(9,),
        in_specs=[
            pl.BlockSpec((120, _D), lambda i: (i, 0)),
            pl.BlockSpec((_D, _W), lambda i: (0, 0)),
            pl.BlockSpec((_D, _W), lambda i: (0, 0)),
        ],
        out_specs=pl.BlockSpec((120, _W), lambda i: (i, 0)),
        out_shape=jax.ShapeDtypeStruct((_NROWS, _W), jnp.int32),
        compiler_params=pltpu.CompilerParams(
            dimension_semantics=("arbitrary",)),
    )(codes, wlo, whi)
    pkb = packed.reshape(-1)

    mesh = plsc.VectorSubcoreMesh(core_axis_name="c", subcore_axis_name="s")
    f = functools.partial(
        pl.kernel,
        mesh=mesh,
        compiler_params=pltpu.CompilerParams(needs_layout_passes=False),
        out_type=jax.ShapeDtypeStruct((_B, _D), jnp.float32),
        scratch_types=[
            pltpu.VMEM((_T * _C,), jnp.int32),
            pltpu.VMEM((_LEVELS * _W,), jnp.int32),
            pltpu.VMEM((_C * _W,), jnp.int32),
            pltpu.VMEM((_T * _W,), jnp.int32),
            pltpu.VMEM((_T * _ROWW,), jnp.int32),
            pltpu.VMEM((_T * _W,), jnp.int32),
            pltpu.VMEM((_T * _W,), jnp.int32),
            pltpu.VMEM((_T * _W,), jnp.int32),
            pltpu.VMEM((_D,), jnp.float32),
            pltpu.SemaphoreType.DMA,
        ],
    )(_tec_body)
    return f(idx.reshape(-1), pkb)


def kernel(input, level_weight, channel_weight, feature_weight):
    return _encode(input, level_weight, channel_weight, feature_weight)
